# baseline (speedup 1.0000x reference)
import functools

import jax
import jax.numpy as jnp
from jax import lax
from jax.experimental import pallas as pl
from jax.experimental.pallas import tpu as pltpu
from jax.experimental.pallas import tpu_sc as plsc

D = 64
S = 128
LANES = 16
SBLK = S // LANES

NC, NS = 2, 16
NW = NC * NS

CHUNK = 256
SPC = CHUNK // S
NB = 2


def _body(x_hbm, tok_hbm, pos_hbm, out_hbm, pos_v, pos_tr, idxs, rows, trs,
          gsems, ssems):
    n_batch = out_hbm.shape[0]
    per_w = n_batch // NW
    n_chunks = per_w // SPC
    wid = lax.axis_index("s") * NC + lax.axis_index("c")
    tok_base = wid * per_w * S
    seq_base = wid * per_w

    pltpu.sync_copy(pos_hbm, pos_v)

    iotas = [jnp.arange(LANES, dtype=jnp.int32) + s0 * LANES
             for s0 in range(SBLK)]
    ones = jnp.ones((LANES,), jnp.int32)
    zeros = jnp.zeros((LANES,), jnp.int32)

    @pl.loop(0, D, init_carry=zeros)
    def build_pos_tr(d, dd):
        for s0 in range(SBLK):
            v = plsc.load_gather(pos_v, [iotas[s0], dd])
            pos_tr[d, pl.ds(s0 * LANES, LANES)] = v
        return dd + ones

    def start_gather(b, g):
        pltpu.sync_copy(x_hbm.at[pl.ds(tok_base + g * CHUNK, CHUNK)], idxs[b])
        pltpu.async_copy(tok_hbm.at[idxs[b]], rows[b], gsems[b])

    def stage(b, g):
        pltpu.make_async_copy(tok_hbm.at[idxs[b]], rows[b], gsems[b]).wait()

        @pl.when(g >= NB)
        def _():
            for q in range(SPC):
                pltpu.make_async_copy(
                    trs[b].at[pl.ds(q * D, D)],
                    out_hbm.at[seq_base], ssems[b]).wait()

        @pl.loop(0, D, init_carry=zeros, unroll=8)
        def tr_step(d, dd):
            for s0 in range(SBLK):
                pv = pos_tr[d, pl.ds(s0 * LANES, LANES)]
                for q in range(SPC):
                    v = plsc.load_gather(
                        rows[b], [iotas[s0] + q * S, dd])
                    trs[b][q * D + d, pl.ds(s0 * LANES, LANES)] = v + pv
            return dd + ones

        g2 = g + NB

        @pl.when(g2 < n_chunks)
        def _():
            start_gather(b, g2)

        for q in range(SPC):
            pltpu.async_copy(
                trs[b].at[pl.ds(q * D, D)],
                out_hbm.at[seq_base + g * SPC + q], ssems[b])

    for b in range(NB):
        start_gather(b, b)

    def outer(gg, _):
        for b in range(NB):
            stage(b, gg * NB + b)
        return ()

    lax.fori_loop(0, n_chunks // NB, outer, ())
    for b in range(NB):
        for q in range(SPC):
            pltpu.make_async_copy(
                trs[b].at[pl.ds(q * D, D)],
                out_hbm.at[seq_base], ssems[b]).wait()


@jax.jit
def _embed(x_flat, token_table, pos_table):
    n_tokens = x_flat.shape[0]
    n_batch = n_tokens // S
    kern = pl.kernel(
        _body,
        out_type=jax.ShapeDtypeStruct((n_batch, D, S), jnp.float32),
        mesh=plsc.VectorSubcoreMesh(
            core_axis_name="c", subcore_axis_name="s",
            num_cores=NC, num_subcores=NS,
        ),
        scratch_types=[
            pltpu.VMEM((S, D), jnp.float32),
            pltpu.VMEM((D, S), jnp.float32),
            [pltpu.VMEM((CHUNK,), jnp.int32) for _ in range(NB)],
            [pltpu.VMEM((CHUNK, D), jnp.float32) for _ in range(NB)],
            [pltpu.VMEM((SPC * D, S), jnp.float32) for _ in range(NB)],
            [pltpu.SemaphoreType.DMA for _ in range(NB)],
            [pltpu.SemaphoreType.DMA for _ in range(NB)],
        ],
        compiler_params=pltpu.CompilerParams(
            use_tc_tiling_on_sc=False, needs_layout_passes=False),
    )
    return kern(x_flat, token_table, pos_table)


def kernel(x, token_table, pos_table):
    b, s = x.shape
    out_t = _embed(x.reshape(b * s), token_table, pos_table)
    return jnp.transpose(out_t, (0, 2, 1))


# speedup vs baseline: 1.5346x; 1.5346x over previous
import functools

import jax
import jax.numpy as jnp
from jax import lax
from jax.experimental import pallas as pl
from jax.experimental.pallas import tpu as pltpu
from jax.experimental.pallas import tpu_sc as plsc

D = 64
S = 128
LANES = 16
SBLK = S // LANES

NC, NS = 2, 16
NW = NC * NS

CHUNK = 256
SPC = CHUNK // S
NB = 2
SP = S + 1


def _body(x_hbm, tok_hbm, pos_hbm, out_hbm, pos_v, idxs, rows, trs,
          gsems, ssems):
    n_batch = out_hbm.shape[0]
    per_w = n_batch // NW
    n_chunks = per_w // SPC
    wid = lax.axis_index("s") * NC + lax.axis_index("c")
    tok_base = wid * per_w * S
    seq_base = wid * per_w

    pltpu.sync_copy(pos_hbm, pos_v)

    ones = jnp.ones((LANES,), jnp.int32)
    zeros = jnp.zeros((LANES,), jnp.int32)

    def start_gather(b, g):
        pltpu.sync_copy(x_hbm.at[pl.ds(tok_base + g * CHUNK, CHUNK)], idxs[b])
        pltpu.async_copy(tok_hbm.at[idxs[b]], rows[b], gsems[b])

    def stage(b, g):
        pltpu.make_async_copy(tok_hbm.at[idxs[b]], rows[b], gsems[b]).wait()

        @pl.when(g >= NB)
        def _():
            for q in range(SPC):
                pltpu.make_async_copy(
                    trs[b].at[pl.ds(q * D, D), pl.ds(0, S)],
                    out_hbm.at[seq_base], ssems[b]).wait()

        row_idx = [[jnp.arange(LANES, dtype=jnp.int32) + q * D + d0 * LANES
                    for d0 in range(D // LANES)] for q in range(SPC)]

        @pl.loop(0, S, init_carry=zeros, unroll=4)
        def s_step(s, ss):
            for d0 in range(D // LANES):
                pv = pos_v[s, pl.ds(d0 * LANES, LANES)]
                for q in range(SPC):
                    v = rows[b][q * S + s, pl.ds(d0 * LANES, LANES)]
                    plsc.store_scatter(trs[b], [row_idx[q][d0], ss], v + pv)
            return ss + ones

        g2 = g + NB

        @pl.when(g2 < n_chunks)
        def _():
            start_gather(b, g2)

        for q in range(SPC):
            pltpu.async_copy(
                trs[b].at[pl.ds(q * D, D), pl.ds(0, S)],
                out_hbm.at[seq_base + g * SPC + q], ssems[b])

    for b in range(NB):
        start_gather(b, b)

    def outer(gg, _):
        for b in range(NB):
            stage(b, gg * NB + b)
        return ()

    lax.fori_loop(0, n_chunks // NB, outer, ())
    for b in range(NB):
        for q in range(SPC):
            pltpu.make_async_copy(
                trs[b].at[pl.ds(q * D, D), pl.ds(0, S)],
                out_hbm.at[seq_base], ssems[b]).wait()


@jax.jit
def _embed(x_flat, token_table, pos_table):
    n_tokens = x_flat.shape[0]
    n_batch = n_tokens // S
    kern = pl.kernel(
        _body,
        out_type=jax.ShapeDtypeStruct((n_batch, D, S), jnp.float32),
        mesh=plsc.VectorSubcoreMesh(
            core_axis_name="c", subcore_axis_name="s",
            num_cores=NC, num_subcores=NS,
        ),
        scratch_types=[
            pltpu.VMEM((S, D), jnp.float32),
            [pltpu.VMEM((CHUNK,), jnp.int32) for _ in range(NB)],
            [pltpu.VMEM((CHUNK, D), jnp.float32) for _ in range(NB)],
            [pltpu.VMEM((SPC * D, SP), jnp.float32) for _ in range(NB)],
            [pltpu.SemaphoreType.DMA for _ in range(NB)],
            [pltpu.SemaphoreType.DMA for _ in range(NB)],
        ],
        compiler_params=pltpu.CompilerParams(
            use_tc_tiling_on_sc=False, needs_layout_passes=False),
    )
    return kern(x_flat, token_table, pos_table)


def kernel(x, token_table, pos_table):
    b, s = x.shape
    out_t = _embed(x.reshape(b * s), token_table, pos_table)
    return jnp.transpose(out_t, (0, 2, 1))
